# baseline (speedup 1.0000x reference)
import functools

import jax
import jax.numpy as jnp
from jax import lax
from jax.experimental import pallas as pl
from jax.experimental.pallas import tpu as pltpu
from jax.experimental.pallas import tpu_sc as plsc

N = 10000
E = 320000
D = 128
DC = 64

NC = 2
NS = 16
K = 128
C = 160
EPT = K * C
E_PAD = NS * EPT
NPAD = 10112
RPT = NPAD // NS
DUMMY = N + 8
_RCHUNKS = [128, 128, 128, 128, 120]
_XROWS = 624

_mesh = plsc.VectorSubcoreMesh(core_axis_name="c", subcore_axis_name="s")


@functools.partial(
    pl.kernel,
    out_type=jax.ShapeDtypeStruct((NC, NPAD, DC), jnp.float32),
    mesh=_mesh,
    scratch_types=[
        pltpu.VMEM_SHARED((N, DC), jnp.float32),
        pltpu.VMEM_SHARED((NPAD, DC), jnp.float32),
        pltpu.VMEM((EPT // 2,), jnp.int32),
        pltpu.VMEM((1, K), jnp.int32),
        pltpu.VMEM((1, K), jnp.int32),
        pltpu.VMEM((K, DC), jnp.float32),
        pltpu.VMEM((K, DC), jnp.float32),
        pltpu.SemaphoreType.DMA,
        pltpu.SemaphoreType.DMA,
        pltpu.SemaphoreType.DMA,
        pltpu.SemaphoreType.DMA,
    ],
)
def _segment_sum_sc(xt_hbm, src_hbm, dst_hbm, zero_hbm, out_hbm,
                    xsp, acc, src_v, dbuf0, dbuf1, buf0, buf1,
                    gsem0, gsem1, dsem0, dsem1):
    c = lax.axis_index("c")
    s = lax.axis_index("s")

    pltpu.sync_copy(xt_hbm.at[c, pl.ds(s * _XROWS, _XROWS)],
                    xsp.at[pl.ds(s * _XROWS, _XROWS)])

    @pl.when(s == 0)
    def _():
        pltpu.sync_copy(xt_hbm.at[c, pl.ds(NS * _XROWS, N - NS * _XROWS)],
                        xsp.at[pl.ds(NS * _XROWS, N - NS * _XROWS)])

    r0 = s * RPT
    pltpu.sync_copy(zero_hbm, buf0)
    off = 0
    for sz in _RCHUNKS:
        pltpu.sync_copy(buf0.at[pl.ds(0, sz)], acc.at[pl.ds(r0 + off, sz)])
        off += sz

    plsc.subcore_barrier()

    def chunk_start(pbase, j, buf, dbuf, gsem, dsem):
        pltpu.async_copy(xsp.at[src_v.at[pl.ds(j * K, K)]], buf, gsem)
        pltpu.async_copy(dst_hbm.at[pl.ds(s * C + pbase + j, 1)], dbuf, dsem)

    def chunk_finish(buf, dbuf, gsem, dsem):
        pltpu.make_async_copy(xsp.at[src_v.at[pl.ds(0, K)]], buf, gsem).wait()
        pltpu.make_async_copy(dst_hbm.at[pl.ds(0, 1)], dbuf, dsem).wait()
        pltpu.sync_copy(buf, acc.at[dbuf.at[0]], add=True)

    CP = C // 2
    for pbase in (0, CP):
        pltpu.sync_copy(src_hbm.at[pl.ds(s * EPT + pbase * K, CP * K)], src_v)
        chunk_start(pbase, 0, buf0, dbuf0, gsem0, dsem0)

        def body(t, carry, pbase=pbase):
            j0 = 2 * t
            j1 = j0 + 1
            chunk_start(pbase, j1, buf1, dbuf1, gsem1, dsem1)
            chunk_finish(buf0, dbuf0, gsem0, dsem0)

            @pl.when(j1 + 1 < CP)
            def _():
                chunk_start(pbase, j1 + 1, buf0, dbuf0, gsem0, dsem0)

            chunk_finish(buf1, dbuf1, gsem1, dsem1)
            return carry

        lax.fori_loop(0, CP // 2, body, 0)
    plsc.subcore_barrier()

    off = 0
    for sz in _RCHUNKS:
        pltpu.sync_copy(acc.at[pl.ds(r0 + off, sz)], buf0.at[pl.ds(0, sz)])
        pltpu.sync_copy(buf0.at[pl.ds(0, sz)], out_hbm.at[c, pl.ds(r0 + off, sz)])
        off += sz


def _mlp_body(x_ref, p0_ref, p1_ref, w1_ref, b1_ref, w2_ref, b2_ref, o_ref):
    agg = jnp.concatenate([p0_ref[0], p1_ref[0]], axis=1)
    sm = x_ref[...] + agg
    h = jnp.dot(sm, w1_ref[...], preferred_element_type=jnp.float32)
    h = jnp.maximum(h + b1_ref[...], 0.0)
    o = jnp.dot(h, w2_ref[...], preferred_element_type=jnp.float32)
    o_ref[...] = jnp.maximum(o + b2_ref[...], 0.0)


_BLK = 1000


def _mlp_tc(x, parts, W1, b1, W2, b2):
    grid = (N // _BLK,)
    return pl.pallas_call(
        _mlp_body,
        grid=grid,
        in_specs=[
            pl.BlockSpec((_BLK, D), lambda i: (i, 0)),
            pl.BlockSpec((1, _BLK, DC), lambda i: (0, i, 0)),
            pl.BlockSpec((1, _BLK, DC), lambda i: (1, i, 0)),
            pl.BlockSpec((D, D), lambda i: (0, 0)),
            pl.BlockSpec((1, D), lambda i: (0, 0)),
            pl.BlockSpec((D, D), lambda i: (0, 0)),
            pl.BlockSpec((1, D), lambda i: (0, 0)),
        ],
        out_specs=pl.BlockSpec((_BLK, D), lambda i: (i, 0)),
        out_shape=jax.ShapeDtypeStruct((N, D), jnp.float32),
    )(x, parts, parts, W1, b1, W2, b2)


def kernel(x, edge_index, W1a, b1a, W1b, b1b, W2a, b2a, W2b, b2b):
    pad = E_PAD - E
    src = jnp.concatenate([edge_index[0], jnp.zeros((pad,), jnp.int32)])
    dst = jnp.concatenate([edge_index[1], jnp.full((pad,), DUMMY, jnp.int32)])
    dst = dst.reshape(NS * C, K)
    zero = jnp.zeros((K, DC), jnp.float32)

    xt = jnp.stack([x[:, :DC], x[:, DC:]], axis=0)
    parts1 = _segment_sum_sc(xt, src, dst, zero)
    h1 = _mlp_tc(x, parts1, W1a, b1a.reshape(1, D), W1b, b1b.reshape(1, D))
    h1t = jnp.stack([h1[:, :DC], h1[:, DC:]], axis=0)
    parts2 = _segment_sum_sc(h1t, src, dst, zero)
    h2 = _mlp_tc(h1, parts2, W2a, b2a.reshape(1, D), W2b, b2b.reshape(1, D))
    return jnp.concatenate([x, h1, h2], axis=1)
